# baseline (speedup 1.0000x reference)
import functools

import jax
import jax.numpy as jnp
from jax import lax
from jax.experimental import pallas as pl
from jax.experimental.pallas import tpu as pltpu
from jax.experimental.pallas import tpu_sc as plsc

XW = 335
OW = 408
TW = 16 * 34 * 15
FW = 20 * 9
P = 34
F = 9
O = 12
NC = 2
NS = 16
NW = NC * NS
L = 16


def _sc_conv(x_flat, tab_flat, flt_flat, B):
    spw = B // NW
    ng = spw // L
    mesh = plsc.VectorSubcoreMesh(core_axis_name="c", subcore_axis_name="s")

    @functools.partial(
        pl.kernel,
        mesh=mesh,
        out_type=jax.ShapeDtypeStruct((B * OW,), jnp.float32),
        scratch_types=[
            pltpu.VMEM((spw * XW,), jnp.float32),
            pltpu.VMEM((spw * OW,), jnp.float32),
            pltpu.VMEM((TW,), jnp.int32),
            pltpu.VMEM((FW,), jnp.float32),
        ],
        compiler_params=pltpu.CompilerParams(needs_layout_passes=False),
    )
    def k(x_hbm, tab_hbm, flt_hbm, out_hbm, xs, outs, tab, flt):
        wid = lax.axis_index("s") * NC + lax.axis_index("c")
        pltpu.sync_copy(tab_hbm, tab)
        pltpu.sync_copy(flt_hbm, flt)
        pltpu.sync_copy(x_hbm.at[pl.ds(wid * (spw * XW), spw * XW)], xs)

        lanes = lax.broadcasted_iota(jnp.int32, (L,), 0)

        def gbody(g, _):
            lane_base = lanes * XW + g * (L * XW)
            out_base = lanes * OW + g * (L * OW)
            len_i = plsc.load_gather(xs, [lane_base]).astype(jnp.int32)
            tab_base = len_i * (P * 15)

            @plsc.parallel_loop(0, P, unroll=2)
            def pbody(p):
                i0 = plsc.load_gather(tab, [tab_base + p * 15])
                i1 = plsc.load_gather(tab, [tab_base + (p * 15 + 1)])
                a = plsc.load_gather(xs, [lane_base + (301 + p)]).astype(jnp.int32)
                kbase = a * F
                kf = [plsc.load_gather(flt, [kbase + f]) for f in range(F)]
                r0 = lane_base + (i0 * 20 - 19)
                r1 = lane_base + (i1 * 20 - 19)
                s = [plsc.load_gather(xs, [r0 + c]) + plsc.load_gather(xs, [r1 + c])
                     for c in range(20)]
                ob = out_base + p * O
                for o in range(O):
                    acc = s[o] * kf[0]
                    for f in range(1, F):
                        acc = acc + s[o + f] * kf[f]
                    plsc.store_scatter(outs, [ob + o], acc)

            return 0

        lax.fori_loop(0, ng, gbody, 0, unroll=False)
        pltpu.sync_copy(outs, out_hbm.at[pl.ds(wid * (spw * OW), spw * OW)])

    return k(x_flat, tab_flat, flt_flat)


def kernel(x, kernel, pocket_table):
    B = x.shape[0]
    out = _sc_conv(
        x.reshape(-1),
        pocket_table.astype(jnp.int32).reshape(-1),
        kernel.reshape(-1),
        B,
    )
    return out.reshape(B, OW)


# speedup vs baseline: 1.1670x; 1.1670x over previous
import functools

import jax
import jax.numpy as jnp
from jax import lax
from jax.experimental import pallas as pl
from jax.experimental.pallas import tpu as pltpu
from jax.experimental.pallas import tpu_sc as plsc

XW = 335
OW = 408
TW = 16 * 34 * 15
FW = 20 * 9
P = 34
F = 9
O = 12
NC = 2
NS = 16
NW = NC * NS
L = 16


def _sc_conv(x_flat, tab_flat, flt_flat, B):
    spw = B // NW
    ng = spw // L
    mesh = plsc.VectorSubcoreMesh(core_axis_name="c", subcore_axis_name="s")

    @functools.partial(
        pl.kernel,
        mesh=mesh,
        out_type=jax.ShapeDtypeStruct((B * OW,), jnp.float32),
        scratch_types=[
            pltpu.VMEM((spw * XW,), jnp.float32),
            pltpu.VMEM((spw * OW,), jnp.float32),
            pltpu.VMEM((TW,), jnp.int32),
            pltpu.VMEM((FW,), jnp.float32),
        ],
        compiler_params=pltpu.CompilerParams(needs_layout_passes=False),
    )
    def k(x_hbm, tab_hbm, flt_hbm, out_hbm, xs, outs, tab, flt):
        wid = lax.axis_index("s") * NC + lax.axis_index("c")
        pltpu.sync_copy(tab_hbm, tab)
        pltpu.sync_copy(flt_hbm, flt)
        pltpu.sync_copy(x_hbm.at[pl.ds(wid * (spw * XW), spw * XW)], xs)

        lanes = lax.broadcasted_iota(jnp.int32, (L,), 0)

        def gbody(g, _):
            lane_base = lanes * XW + g * (L * XW)
            out_base = lanes * OW + g * (L * OW)
            len_i = plsc.load_gather(xs, [lane_base]).astype(jnp.int32)
            tab_base = len_i * (P * 15)

            def pbody(p, _):
                i0 = plsc.load_gather(tab, [tab_base + p * 15])
                i1 = plsc.load_gather(tab, [tab_base + (p * 15 + 1)])
                a = plsc.load_gather(xs, [lane_base + (301 + p)]).astype(jnp.int32)
                kbase = a * F
                kf = [plsc.load_gather(flt, [kbase + f]) for f in range(F)]
                r0 = lane_base + (i0 * 20 - 19)
                r1 = lane_base + (i1 * 20 - 19)
                s = [plsc.load_gather(xs, [r0 + c]) + plsc.load_gather(xs, [r1 + c])
                     for c in range(20)]
                ob = out_base + p * O
                for o in range(O):
                    acc = s[o] * kf[0]
                    for f in range(1, F):
                        acc = acc + s[o + f] * kf[f]
                    plsc.store_scatter(outs, [ob + o], acc)
                return 0

            lax.fori_loop(0, P, pbody, 0, unroll=2)
            return 0

        lax.fori_loop(0, ng, gbody, 0, unroll=False)
        pltpu.sync_copy(outs, out_hbm.at[pl.ds(wid * (spw * OW), spw * OW)])

    return k(x_flat, tab_flat, flt_flat)


def kernel(x, kernel, pocket_table):
    B = x.shape[0]
    out = _sc_conv(
        x.reshape(-1),
        pocket_table.astype(jnp.int32).reshape(-1),
        kernel.reshape(-1),
        B,
    )
    return out.reshape(B, OW)


# speedup vs baseline: 1.1678x; 1.0006x over previous
import functools

import jax
import jax.numpy as jnp
from jax import lax
from jax.experimental import pallas as pl
from jax.experimental.pallas import tpu as pltpu
from jax.experimental.pallas import tpu_sc as plsc

XW = 335
OW = 408
MAXL = 16
ALPHA = 20
P = 34
S15 = 15
F = 9
O = 12
NC = 2
NS = 16
NW = NC * NS
L = 16


def _sc_conv(xf, tabf, fltf, B):
    spw = B // NW
    ng = spw // L
    mesh = plsc.VectorSubcoreMesh(core_axis_name="c", subcore_axis_name="s")

    @functools.partial(
        pl.kernel,
        mesh=mesh,
        out_type=jax.ShapeDtypeStruct((B * OW,), jnp.float32),
        scratch_types=[
            pltpu.VMEM((spw * XW,), jnp.float32),
            pltpu.VMEM((spw * OW,), jnp.float32),
            pltpu.VMEM((MAXL * P * S15,), jnp.int32),
            pltpu.VMEM((ALPHA * F,), jnp.float32),
        ],
        compiler_params=pltpu.CompilerParams(needs_layout_passes=False),
    )
    def k(x_hbm, tab_hbm, flt_hbm, out_hbm, xs, outs, tabs, flts):
        wid = lax.axis_index("s") * NC + lax.axis_index("c")
        pltpu.sync_copy(tab_hbm, tabs)
        pltpu.sync_copy(flt_hbm, flts)
        pltpu.sync_copy(x_hbm.at[pl.ds(wid * (spw * XW), spw * XW)], xs)

        lanes = lax.broadcasted_iota(jnp.int32, (L,), 0)

        def gbody(g, _):
            rb = (lanes + g * L) * XW
            ob = (lanes + g * L) * OW
            len_i = plsc.load_gather(xs, [rb]).astype(jnp.int32)
            tb = len_i * (P * S15)

            def pbody(p, _):
                i0 = plsc.load_gather(tabs, [tb + p * S15])
                i1 = plsc.load_gather(tabs, [tb + (p * S15 + 1)])
                a = plsc.load_gather(xs, [rb + (301 + p)]).astype(jnp.int32) * F
                kf = [plsc.load_gather(flts, [a + f]) for f in range(F)]
                c0 = rb + i0 * 20 - 19
                c1 = rb + i1 * 20 - 19
                s = [plsc.load_gather(xs, [c0 + c]) + plsc.load_gather(xs, [c1 + c])
                     for c in range(20)]
                po = ob + p * O
                for o in range(O):
                    acc = s[o] * kf[0]
                    for f in range(1, F):
                        acc = acc + s[o + f] * kf[f]
                    plsc.store_scatter(outs, [po + o], acc)
                return 0

            lax.fori_loop(0, P, pbody, 0, unroll=2)
            return 0

        lax.fori_loop(0, ng, gbody, 0, unroll=False)
        pltpu.sync_copy(outs, out_hbm.at[pl.ds(wid * (spw * OW), spw * OW)])

    return k(xf, tabf, fltf)


def kernel(x, kernel, pocket_table):
    B = x.shape[0]
    out = _sc_conv(
        x.reshape(-1),
        pocket_table.astype(jnp.int32).reshape(-1),
        kernel.reshape(-1),
        B,
    )
    return out.reshape(B, OW)
